# speedup vs baseline: 5.4194x; 5.4194x over previous
import functools

import jax
import jax.numpy as jnp
from jax import lax
from jax.experimental import pallas as pl
from jax.experimental.pallas import tpu as pltpu
from jax.experimental.pallas import tpu_sc as plsc

N_NODES = 10000
DIM = 128
N_EDGES = 320000
KAPPA_C = 0.95
EPS_C = 1e-05

NUM_CORES = 2
NUM_SUBCORES = 16
NUM_WORKERS = NUM_CORES * NUM_SUBCORES

ACC_ROWS = 10240
ROWS_PER_TILE = ACC_ROWS // NUM_SUBCORES
EDGES_PER_WORKER = N_EDGES // NUM_WORKERS
CHUNK = 80
CHUNKS_PER_WORKER = EDGES_PER_WORKER // CHUNK

_mesh = plsc.VectorSubcoreMesh(core_axis_name="c", subcore_axis_name="s")


@functools.partial(
    pl.kernel,
    mesh=_mesh,
    out_type=jax.ShapeDtypeStruct((NUM_CORES, ACC_ROWS, DIM), jnp.float32),
    scratch_types=[
        pltpu.VMEM_SHARED((ACC_ROWS, DIM), jnp.float32),
        pltpu.VMEM((CHUNK,), jnp.int32),
        pltpu.VMEM((CHUNK,), jnp.int32),
        pltpu.VMEM((CHUNK, DIM), jnp.float32),
        pltpu.SemaphoreType.DMA,
    ],
)
def _sc_aggregate(src_hbm, dst_hbm, emb_hbm, zeros_hbm, out_hbm,
                  acc, src_v, dst_v, rows_v, sem):
    c = lax.axis_index("c")
    s = lax.axis_index("s")
    wid = c * NUM_SUBCORES + s

    row0 = s * ROWS_PER_TILE
    pltpu.sync_copy(zeros_hbm.at[pl.ds(row0, ROWS_PER_TILE)],
                    acc.at[pl.ds(row0, ROWS_PER_TILE)])
    plsc.subcore_barrier()

    ebase = wid * EDGES_PER_WORKER

    def body(i, _):
        base = ebase + i * CHUNK
        pltpu.sync_copy(src_hbm.at[pl.ds(base, CHUNK)], src_v)
        pltpu.sync_copy(dst_hbm.at[pl.ds(base, CHUNK)], dst_v)
        pltpu.async_copy(emb_hbm.at[src_v], rows_v, sem).wait()
        pltpu.sync_copy(rows_v, acc.at[dst_v], add=True)
        return ()

    lax.fori_loop(0, CHUNKS_PER_WORKER, body, ())
    plsc.subcore_barrier()

    pltpu.sync_copy(acc.at[pl.ds(row0, ROWS_PER_TILE)],
                    out_hbm.at[c, pl.ds(row0, ROWS_PER_TILE)])


ROW_BLOCK = 1000
GRID = N_NODES // ROW_BLOCK


def _tc_body(p_ref, f_ref, pre_ref, o_ref):
    f = f_ref[...]
    w = lax.dot_general(f, f, (((0,), (0,)), ((), ())),
                        preferred_element_type=jnp.float32)
    w = w / (jnp.sqrt(jnp.sum(w * w)) + EPS_C)
    agg = p_ref[0] + p_ref[1]
    o_ref[...] = KAPPA_C * lax.dot_general(
        agg, w, (((1,), (0,)), ((), ())),
        preferred_element_type=jnp.float32) + pre_ref[...]


_tc_call = pl.pallas_call(
    _tc_body,
    grid=(GRID,),
    in_specs=[
        pl.BlockSpec((NUM_CORES, ROW_BLOCK, DIM), lambda i: (0, i, 0)),
        pl.BlockSpec((DIM, DIM), lambda i: (0, 0)),
        pl.BlockSpec((ROW_BLOCK, DIM), lambda i: (i, 0)),
    ],
    out_specs=pl.BlockSpec((ROW_BLOCK, DIM), lambda i: (i, 0)),
    out_shape=jax.ShapeDtypeStruct((N_NODES, DIM), jnp.float32),
)


def kernel(features, edge_index, embeddings, F_param, pretrained_embeddings):
    del features
    dst = edge_index[0]
    src = edge_index[1]
    zeros = jnp.zeros((ACC_ROWS, DIM), jnp.float32)
    partials = _sc_aggregate(src, dst, embeddings, zeros)
    return _tc_call(partials, F_param, pretrained_embeddings)
